# baseline (speedup 1.0000x reference)
import functools

import jax
import jax.numpy as jnp
from jax import lax
from jax.experimental import pallas as pl
from jax.experimental.pallas import tpu as pltpu
from jax.experimental.pallas import tpu_sc as plsc

N = 10000
E = 320000
D = 128
R = 16

NC = 2
NS = 16
NW = NC * NS
C = 128
CPAD = 2560
EPAD = CPAD * C
CPT = CPAD // NW
CPT_H = CPT // 2
NPAD = 10240
ROWS_PER_TILE = NPAD // NS
ZROWS = 128

BN = 1000


def _matmul_body(h_ref, w_ref, z_ref):
    z_ref[0] = jnp.dot(h_ref[...], w_ref[0], preferred_element_type=jnp.float32)


def _tc_transform(h, wcat):
    nt = N // BN
    rp = R + 1
    return pl.pallas_call(
        _matmul_body,
        grid=(nt, rp),
        in_specs=[
            pl.BlockSpec((BN, D), lambda i, r: (i, 0)),
            pl.BlockSpec((1, D, D), lambda i, r: (r, 0, 0)),
        ],
        out_specs=pl.BlockSpec((1, BN, D), lambda i, r: (r, i, 0)),
        out_shape=jax.ShapeDtypeStruct((rp, N, D), jnp.float32),
    )(h, wcat)


def _fill_rows(ref, nrows, val):
    def _row(i, carry):
        for j in range(D // 16):
            ref[i, pl.ds(16 * j, 16)] = jnp.full((16,), val, jnp.float32)
        return carry

    lax.fori_loop(0, nrows, _row, 0)


def _zero_shared(acc_sh, zrows, s):
    for k in range(ROWS_PER_TILE // ZROWS):
        pltpu.sync_copy(
            zrows.at[pl.ds(0, ZROWS)],
            acc_sh.at[pl.ds(s * ROWS_PER_TILE + k * ZROWS, ZROWS)],
        )


def _writeback(acc_sh, stage, acc_out, c, s):
    row0 = s * ROWS_PER_TILE
    for k in range(ROWS_PER_TILE // ZROWS):
        r0 = row0 + k * ZROWS
        pltpu.sync_copy(acc_sh.at[pl.ds(r0, ZROWS)], stage)
        pltpu.sync_copy(stage, acc_out.at[c, pl.ds(r0, ZROWS)])


def _sc_body(z_ref, src_ref, dst_ref, et_ref, acc_out,
             src_st, et_st, dst_all, rid_all, rows_a, rows_b,
             acc_sh, gsa, gsb, ssa, ssb):
    c = lax.axis_index("c")
    s = lax.axis_index("s")
    wid = s * NC + c

    _fill_rows(rows_a, C, 0.0)
    _zero_shared(acc_sh, rows_a, s)

    def _gather(j, buf, sem):
        return pltpu.async_copy(z_ref.at[rid_all.at[j]], buf, sem)

    def _gather_wait(j, buf, sem):
        pltpu.make_async_copy(z_ref.at[rid_all.at[j]], buf, sem).wait()

    def _scat(j, buf, sem):
        return pltpu.async_copy(buf, acc_sh.at[dst_all.at[j]], sem, add=True)

    def _scat_wait(j, buf, sem):
        pltpu.make_async_copy(buf, acc_sh.at[dst_all.at[j]], sem).wait()

    for half in range(2):
        base = wid * CPT + half * CPT_H
        pltpu.sync_copy(dst_ref.at[pl.ds(base, CPT_H)], dst_all)

        def _rid(i, carry):
            pltpu.sync_copy(src_ref.at[pl.ds(base + 8 * i, 8)], src_st)
            pltpu.sync_copy(et_ref.at[pl.ds(base + 8 * i, 8)], et_st)
            for b in range(8):
                for j in range(C // 16):
                    sl = pl.ds(16 * j, 16)
                    rid_all[8 * i + b, sl] = et_st[b, sl] * N + src_st[b, sl]
            return carry

        lax.fori_loop(0, CPT_H // 8, _rid, 0)

        def _chunk(j, carry):
            _gather(j, rows_a, gsa).wait()
            _scat(j, rows_a, ssa)
            _scat_wait(j, rows_a, ssa)
            return carry

        lax.fori_loop(0, CPT_H, _chunk, 0)

    plsc.subcore_barrier()
    _writeback(acc_sh, rows_a, acc_out, c, s)


def _deg_body(dst_ref, deg_out, dst_all, ones_v, stage, deg_sh, sem):
    c = lax.axis_index("c")
    s = lax.axis_index("s")
    wid = s * NC + c

    _fill_rows(ones_v, C, 0.0)
    _zero_shared(deg_sh, ones_v, s)
    _fill_rows(ones_v, C, 1.0)

    base = wid * CPT
    pltpu.sync_copy(dst_ref.at[pl.ds(base, CPT)], dst_all)
    plsc.subcore_barrier()

    def _batch(i, carry):
        for j in range(8):
            pltpu.async_copy(ones_v, deg_sh.at[dst_all.at[8 * i + j]], sem,
                             add=True)
        for j in range(8):
            pltpu.make_async_copy(ones_v, deg_sh.at[dst_all.at[8 * i + j]],
                                  sem).wait()
        return carry

    lax.fori_loop(0, CPT // 8, _batch, 0)

    plsc.subcore_barrier()
    _writeback(deg_sh, stage, deg_out, c, s)


def _sc_mesh():
    return plsc.VectorSubcoreMesh(
        core_axis_name="c", subcore_axis_name="s",
        num_cores=NC, num_subcores=NS,
    )


@functools.lru_cache(maxsize=None)
def _make_sc():
    return pl.kernel(
        _sc_body,
        out_type=[jax.ShapeDtypeStruct((NC, NPAD, D), jnp.float32)],
        mesh=_sc_mesh(),
        scratch_types=[
            pltpu.VMEM((8, C), jnp.int32),
            pltpu.VMEM((8, C), jnp.int32),
            pltpu.VMEM((CPT_H, C), jnp.int32),
            pltpu.VMEM((CPT_H, C), jnp.int32),
            pltpu.VMEM((C, D), jnp.float32),
            pltpu.VMEM((C, D), jnp.float32),
            pltpu.VMEM_SHARED((NPAD, D), jnp.float32),
            pltpu.SemaphoreType.DMA,
            pltpu.SemaphoreType.DMA,
            pltpu.SemaphoreType.DMA,
            pltpu.SemaphoreType.DMA,
        ],
    )


@functools.lru_cache(maxsize=None)
def _make_deg():
    return pl.kernel(
        _deg_body,
        out_type=[jax.ShapeDtypeStruct((NC, NPAD, D), jnp.float32)],
        mesh=_sc_mesh(),
        scratch_types=[
            pltpu.VMEM((CPT, C), jnp.int32),
            pltpu.VMEM((C, D), jnp.float32),
            pltpu.VMEM((ZROWS, D), jnp.float32),
            pltpu.VMEM_SHARED((NPAD, D), jnp.float32),
            pltpu.SemaphoreType.DMA,
        ],
    )


def _combine_body(acc_ref, degp_ref, z_ref, out_ref):
    d = jnp.maximum(degp_ref[0, :, 0:1] + degp_ref[1, :, 0:1], 1.0)
    agg = acc_ref[0] + acc_ref[1]
    out_ref[...] = jnp.maximum(agg / d + z_ref[0], 0.0)


def _combine(acc, degp, z):
    nt = N // BN
    return pl.pallas_call(
        _combine_body,
        grid=(nt,),
        in_specs=[
            pl.BlockSpec((NC, BN, D), lambda i: (0, i, 0)),
            pl.BlockSpec((NC, BN, D), lambda i: (0, i, 0)),
            pl.BlockSpec((1, BN, D), lambda i: (R, i, 0)),
        ],
        out_specs=pl.BlockSpec((BN, D), lambda i: (i, 0)),
        out_shape=jax.ShapeDtypeStruct((N, D), jnp.float32),
    )(acc, degp, z)


def kernel(x, edge_index, edge_type, W, W_self):
    src = edge_index[0]
    dst = edge_index[1]
    et = edge_type
    pad = EPAD - E
    src2d = jnp.concatenate([src, jnp.zeros((pad,), jnp.int32)]).reshape(CPAD, C)
    dst2d = jnp.concatenate([dst, jnp.full((pad,), N, jnp.int32)]).reshape(CPAD, C)
    et2d = jnp.concatenate([et, jnp.zeros((pad,), jnp.int32)]).reshape(CPAD, C)
    (degp,) = _make_deg()(dst2d)
    h = x
    for l in range(2):
        wcat = jnp.concatenate([W[l], W_self[l][None]], axis=0)
        z = _tc_transform(h, wcat)
        zflat = z.reshape(((R + 1) * N, D))
        (acc,) = _make_sc()(zflat, src2d, dst2d, et2d)
        h = _combine(acc, degp, z)
    return h


# speedup vs baseline: 1.2630x; 1.2630x over previous
import functools

import jax
import jax.numpy as jnp
from jax import lax
from jax.experimental import pallas as pl
from jax.experimental.pallas import tpu as pltpu
from jax.experimental.pallas import tpu_sc as plsc

N = 10000
E = 320000
D = 128
R = 16

NC = 2
NS = 16
NW = NC * NS
C = 128
CPAD = 2560
EPAD = CPAD * C
CPT = CPAD // NW
CPT_H = CPT // 2
NPAD = 10240
ROWS_PER_TILE = NPAD // NS
ZROWS = 128

BN = 1000


def _matmul_body(h_ref, w_ref, z_ref):
    z_ref[0] = jnp.dot(h_ref[...], w_ref[0], preferred_element_type=jnp.float32)


def _tc_transform(h, wcat):
    nt = N // BN
    rp = R + 1
    return pl.pallas_call(
        _matmul_body,
        grid=(nt, rp),
        in_specs=[
            pl.BlockSpec((BN, D), lambda i, r: (i, 0)),
            pl.BlockSpec((1, D, D), lambda i, r: (r, 0, 0)),
        ],
        out_specs=pl.BlockSpec((1, BN, D), lambda i, r: (r, i, 0)),
        out_shape=jax.ShapeDtypeStruct((rp, N, D), jnp.float32),
    )(h, wcat)


def _fill_rows(ref, nrows, val):
    def _row(i, carry):
        for j in range(D // 16):
            ref[i, pl.ds(16 * j, 16)] = jnp.full((16,), val, jnp.float32)
        return carry

    lax.fori_loop(0, nrows, _row, 0)


def _zero_shared(acc_sh, zrows, s):
    for k in range(ROWS_PER_TILE // ZROWS):
        pltpu.sync_copy(
            zrows.at[pl.ds(0, ZROWS)],
            acc_sh.at[pl.ds(s * ROWS_PER_TILE + k * ZROWS, ZROWS)],
        )


def _writeback(acc_sh, stage, acc_out, c, s):
    row0 = s * ROWS_PER_TILE
    for k in range(ROWS_PER_TILE // ZROWS):
        r0 = row0 + k * ZROWS
        pltpu.sync_copy(acc_sh.at[pl.ds(r0, ZROWS)], stage)
        pltpu.sync_copy(stage, acc_out.at[c, pl.ds(r0, ZROWS)])


def _sc_body(z_ref, src_ref, dst_ref, et_ref, acc_out,
             src_a, et_a, dst_a, rid_a, src_b, et_b, dst_b, rid_b,
             rows_a, rows_b, acc_sh, gsa, gsb, ssa, ssb):
    c = lax.axis_index("c")
    s = lax.axis_index("s")
    wid = s * NC + c

    _fill_rows(rows_a, C, 0.0)
    _zero_shared(acc_sh, rows_a, s)
    plsc.subcore_barrier()

    def _loadidx(i, sv, ev, dv, rv):
        base = (wid + i * NW) * C
        pltpu.sync_copy(src_ref.at[pl.ds(base, C)], sv)
        pltpu.sync_copy(et_ref.at[pl.ds(base, C)], ev)
        pltpu.sync_copy(dst_ref.at[pl.ds(base, C)], dv)
        for j in range(C // 16):
            sl = pl.ds(16 * j, 16)
            rv[sl] = ev[sl] * N + sv[sl]

    def _gather(rv, buf, sem):
        return pltpu.async_copy(z_ref.at[rv], buf, sem)

    def _gather_wait(rv, buf, sem):
        pltpu.make_async_copy(z_ref.at[rv], buf, sem).wait()

    def _scat(dv, buf, sem):
        return pltpu.async_copy(buf, acc_sh.at[dv], sem, add=True)

    def _scat_wait(dv, buf, sem):
        pltpu.make_async_copy(buf, acc_sh.at[dv], sem).wait()

    _loadidx(0, src_a, et_a, dst_a, rid_a)
    _gather(rid_a, rows_a, gsa)

    def _pair(i, carry):
        _loadidx(2 * i + 1, src_b, et_b, dst_b, rid_b)
        _gather_wait(rid_a, rows_a, gsa)
        _scat(dst_a, rows_a, ssa)
        _gather(rid_b, rows_b, gsb)
        _scat_wait(dst_a, rows_a, ssa)
        _loadidx(2 * i + 2, src_a, et_a, dst_a, rid_a)
        _gather(rid_a, rows_a, gsa)
        _gather_wait(rid_b, rows_b, gsb)
        _scat(dst_b, rows_b, ssb)
        _scat_wait(dst_b, rows_b, ssb)
        return carry

    lax.fori_loop(0, CPT // 2 - 1, _pair, 0)

    _gather_wait(rid_a, rows_a, gsa)
    _scat(dst_a, rows_a, ssa)
    _scat_wait(dst_a, rows_a, ssa)
    _loadidx(CPT - 1, src_b, et_b, dst_b, rid_b)
    _gather(rid_b, rows_b, gsb).wait()
    _scat(dst_b, rows_b, ssb)
    _scat_wait(dst_b, rows_b, ssb)

    plsc.subcore_barrier()
    _writeback(acc_sh, rows_a, acc_out, c, s)


def _deg_body(dst_ref, deg_out, dst_all, ones_v, stage, deg_sh, sem):
    c = lax.axis_index("c")
    s = lax.axis_index("s")
    wid = s * NC + c

    _fill_rows(ones_v, C, 0.0)
    _zero_shared(deg_sh, ones_v, s)
    _fill_rows(ones_v, C, 1.0)

    base = wid * CPT
    pltpu.sync_copy(dst_ref.at[pl.ds(base, CPT)], dst_all)
    plsc.subcore_barrier()

    def _batch(i, carry):
        for j in range(8):
            pltpu.async_copy(ones_v, deg_sh.at[dst_all.at[8 * i + j]], sem,
                             add=True)
        for j in range(8):
            pltpu.make_async_copy(ones_v, deg_sh.at[dst_all.at[8 * i + j]],
                                  sem).wait()
        return carry

    lax.fori_loop(0, CPT // 8, _batch, 0)

    plsc.subcore_barrier()
    _writeback(deg_sh, stage, deg_out, c, s)


def _sc_mesh():
    return plsc.VectorSubcoreMesh(
        core_axis_name="c", subcore_axis_name="s",
        num_cores=NC, num_subcores=NS,
    )


@functools.lru_cache(maxsize=None)
def _make_sc():
    return pl.kernel(
        _sc_body,
        out_type=[jax.ShapeDtypeStruct((NC, NPAD, D), jnp.float32)],
        mesh=_sc_mesh(),
        scratch_types=[
            pltpu.VMEM((C,), jnp.int32),
            pltpu.VMEM((C,), jnp.int32),
            pltpu.VMEM((C,), jnp.int32),
            pltpu.VMEM((C,), jnp.int32),
            pltpu.VMEM((C,), jnp.int32),
            pltpu.VMEM((C,), jnp.int32),
            pltpu.VMEM((C,), jnp.int32),
            pltpu.VMEM((C,), jnp.int32),
            pltpu.VMEM((C, D), jnp.float32),
            pltpu.VMEM((C, D), jnp.float32),
            pltpu.VMEM_SHARED((NPAD, D), jnp.float32),
            pltpu.SemaphoreType.DMA,
            pltpu.SemaphoreType.DMA,
            pltpu.SemaphoreType.DMA,
            pltpu.SemaphoreType.DMA,
        ],
    )


@functools.lru_cache(maxsize=None)
def _make_deg():
    return pl.kernel(
        _deg_body,
        out_type=[jax.ShapeDtypeStruct((NC, NPAD, D), jnp.float32)],
        mesh=_sc_mesh(),
        scratch_types=[
            pltpu.VMEM((CPT, C), jnp.int32),
            pltpu.VMEM((C, D), jnp.float32),
            pltpu.VMEM((ZROWS, D), jnp.float32),
            pltpu.VMEM_SHARED((NPAD, D), jnp.float32),
            pltpu.SemaphoreType.DMA,
        ],
    )


def _combine_body(acc_ref, degp_ref, z_ref, out_ref):
    d = jnp.maximum(degp_ref[0, :, 0:1] + degp_ref[1, :, 0:1], 1.0)
    agg = acc_ref[0] + acc_ref[1]
    out_ref[...] = jnp.maximum(agg / d + z_ref[0], 0.0)


def _combine(acc, degp, z):
    nt = N // BN
    return pl.pallas_call(
        _combine_body,
        grid=(nt,),
        in_specs=[
            pl.BlockSpec((NC, BN, D), lambda i: (0, i, 0)),
            pl.BlockSpec((NC, BN, D), lambda i: (0, i, 0)),
            pl.BlockSpec((1, BN, D), lambda i: (R, i, 0)),
        ],
        out_specs=pl.BlockSpec((BN, D), lambda i: (i, 0)),
        out_shape=jax.ShapeDtypeStruct((N, D), jnp.float32),
    )(acc, degp, z)


def kernel(x, edge_index, edge_type, W, W_self):
    src = edge_index[0]
    dst = edge_index[1]
    et = edge_type
    pad = EPAD - E
    src1 = jnp.concatenate([src, jnp.zeros((pad,), jnp.int32)])
    dst1 = jnp.concatenate([dst, jnp.full((pad,), N, jnp.int32)])
    et1 = jnp.concatenate([et, jnp.zeros((pad,), jnp.int32)])
    (degp,) = _make_deg()(dst1.reshape(CPAD, C))
    h = x
    for l in range(2):
        wcat = jnp.concatenate([W[l], W_self[l][None]], axis=0)
        z = _tc_transform(h, wcat)
        zflat = z.reshape(((R + 1) * N, D))
        (acc,) = _make_sc()(zflat, src1, dst1, et1)
        h = _combine(acc, degp, z)
    return h
